# baseline (speedup 1.0000x reference)
import functools

import jax
import jax.numpy as jnp
from jax import lax
from jax.experimental import pallas as pl
from jax.experimental.pallas import tpu as pltpu
from jax.experimental.pallas import tpu_sc as plsc


def _make_sc_gather(vocab, d, n, nw, chunk):
    b_per_w = n // nw
    assert chunk % 8 == 0 and chunk <= 128
    offs = list(range(0, b_per_w, chunk))
    sizes = [min(chunk, b_per_w - o) for o in offs]
    assert all(s % 8 == 0 for s in sizes)
    mesh = plsc.VectorSubcoreMesh(core_axis_name="c", subcore_axis_name="s")

    @functools.partial(
        pl.kernel,
        mesh=mesh,
        out_type=jax.ShapeDtypeStruct((n, d), jnp.float32),
        scratch_types=[
            pltpu.VMEM((b_per_w,), jnp.int32),
            pltpu.VMEM((chunk, d), jnp.float32),
            pltpu.VMEM((chunk, d), jnp.float32),
            pltpu.SemaphoreType.DMA,
            pltpu.SemaphoreType.DMA,
        ],
    )
    def gather_kernel(table_hbm, idx_hbm, out_hbm, idx_v, rows_a, rows_b, sem_a, sem_b):
        wid = lax.axis_index("s") * 2 + lax.axis_index("c")
        base = wid * b_per_w
        pltpu.sync_copy(idx_hbm.at[pl.ds(base, b_per_w)], idx_v)
        rows = (rows_a, rows_b)
        sems = (sem_a, sem_b)
        pltpu.async_copy(
            table_hbm.at[idx_v.at[pl.ds(0, sizes[0])]],
            rows_a.at[pl.ds(0, sizes[0])],
            sem_a,
        )
        for c in range(len(offs)):
            if c + 1 < len(offs):
                pltpu.async_copy(
                    table_hbm.at[idx_v.at[pl.ds(offs[c + 1], sizes[c + 1])]],
                    rows[(c + 1) % 2].at[pl.ds(0, sizes[c + 1])],
                    sems[(c + 1) % 2],
                )
            pltpu.make_async_copy(
                table_hbm.at[idx_v.at[pl.ds(offs[c], sizes[c])]],
                rows[c % 2].at[pl.ds(0, sizes[c])],
                sems[c % 2],
            ).wait()
            pltpu.sync_copy(
                rows[c % 2].at[pl.ds(0, sizes[c])],
                out_hbm.at[pl.ds(base + offs[c], sizes[c])],
            )

    return gather_kernel


def _mlp_body(rows, l, emb_ref, w1_ref, b1_ref, w2_ref, b2_ref, out_ref):
    emb = emb_ref[...]
    h = jnp.dot(emb, w1_ref[...], preferred_element_type=jnp.float32)
    h = jnp.maximum(h + b1_ref[...], 0.0)
    out = jnp.dot(h, w2_ref[...], preferred_element_type=jnp.float32)
    out = out + b2_ref[...]
    out_ref[...] = out.reshape(rows, l, out.shape[-1])


def _mlp_tc(emb, W1, b1, W2, b2, b_rows, l, block_rows):
    n, d = emb.shape
    vocab = W2.shape[1]
    block_t = block_rows * l
    return pl.pallas_call(
        functools.partial(_mlp_body, block_rows, l),
        grid=(b_rows // block_rows,),
        in_specs=[
            pl.BlockSpec((block_t, d), lambda i: (i, 0)),
            pl.BlockSpec((d, d), lambda i: (0, 0)),
            pl.BlockSpec((1, d), lambda i: (0, 0)),
            pl.BlockSpec((d, vocab), lambda i: (0, 0)),
            pl.BlockSpec((1, vocab), lambda i: (0, 0)),
        ],
        out_specs=pl.BlockSpec((block_rows, l, vocab), lambda i: (i, 0, 0)),
        out_shape=jax.ShapeDtypeStruct((b_rows, l, vocab), jnp.float32),
    )(emb, W1, b1.reshape(1, d), W2, b2.reshape(1, vocab))


def kernel(x, emb_table, W1, b1, W2, b2):
    b, l = x.shape
    vocab, d = emb_table.shape
    n = b * l
    nw = 32
    gather = _make_sc_gather(vocab, d, n, nw, chunk=128)
    emb = gather(emb_table, x.reshape(n))
    return _mlp_tc(emb, W1, b1, W2, b2, b, l, block_rows=64)
